# baseline (speedup 1.0000x reference)
import functools

import jax
import jax.numpy as jnp
from jax import lax
from jax.experimental import pallas as pl
from jax.experimental.pallas import tpu as pltpu
from jax.experimental.pallas import tpu_sc as plsc

_N = 32 * 3 * 512 * 512
_NC, _NS, _L = 2, 16, 16
_NW = _NC * _NS
_HB = 16

_FCH_R = 64
_FCHUNK = _FCH_R * 512
_NFCH = _HB * 3 * (512 // _FCH_R) // _NW

_QCH_R = 128
_QCHUNK = _QCH_R * 512
_NQCH = _HB * 3 * (512 // _QCH_R) // _NW


_sc_mesh = plsc.VectorSubcoreMesh(core_axis_name="c", subcore_axis_name="s")
_sc_params = pltpu.CompilerParams(
    needs_layout_passes=False, use_tc_tiling_on_sc=True
)


def _hist_epilogue(hist2, histt, out_hbm, wid, lane):
    @plsc.parallel_loop(0, _L * 256, step=_L, unroll=4)
    def _tr_body(j):
        lane_out = jnp.right_shift(j, 8)
        bin_base = j & 255
        src = jnp.left_shift(bin_base + lane, 4) | lane_out
        histt[lane_out, pl.ds(bin_base, _L)] = plsc.load_gather(hist2, [src])

    pltpu.sync_copy(histt, out_hbm.at[wid])


def _zero_hist(hist2):
    zero = jnp.zeros((_L,), jnp.int32)

    @plsc.parallel_loop(0, 256 * _L, step=_L)
    def _zero_body(r):
        hist2[pl.ds(r, _L)] = zero


@functools.partial(
    pl.kernel,
    out_type=jax.ShapeDtypeStruct((_NW, _L, 256), jnp.int32),
    mesh=_sc_mesh,
    compiler_params=_sc_params,
    scratch_types=[
        pltpu.VMEM((_FCH_R, 512), jnp.float32),
        pltpu.VMEM((_FCH_R, 512), jnp.float32),
        pltpu.VMEM((256 * _L,), jnp.int32),
        pltpu.VMEM((_L, 256), jnp.int32),
        pltpu.SemaphoreType.DMA,
        pltpu.SemaphoreType.DMA,
    ],
)
def _sc_hist_f32(x_hbm, out_hbm, buf0, buf1, hist2, histt, sem0, sem1):
    wid = lax.axis_index("s") * _NC + lax.axis_index("c")
    _zero_hist(hist2)

    bufs = (buf0, buf1)
    sems = (sem0, sem1)
    copies = [None, None]
    lane = lax.broadcasted_iota(jnp.int32, (_L,), 0)
    one = jnp.ones((_L,), jnp.int32)

    def _src(c):
        g = wid * _NFCH + c
        b = g // 24
        ch = (g % 24) // 8
        rb = g % 8
        return x_hbm.at[b, ch, pl.ds(rb * _FCH_R, _FCH_R), :]

    copies[0] = pltpu.async_copy(_src(0), buf0, sem0)
    for c in range(_NFCH):
        if c + 1 < _NFCH:
            nxt = (c + 1) % 2
            copies[nxt] = pltpu.async_copy(_src(c + 1), bufs[nxt], sems[nxt])
        copies[c % 2].wait()
        cur = bufs[c % 2]

        @plsc.parallel_loop(0, _FCHUNK, step=_L, unroll=8)
        def _chunk_body(i):
            x = cur[jnp.right_shift(i, 9), pl.ds(i & 511, _L)]
            idx = (x * 256.0).astype(jnp.int32)
            slot = jnp.left_shift(idx, 4) | lane
            plsc.addupdate_scatter(hist2, [slot], one)

    _hist_epilogue(hist2, histt, out_hbm, wid, lane)


@functools.partial(
    pl.kernel,
    out_type=jax.ShapeDtypeStruct((_NW, _L, 256), jnp.int32),
    mesh=_sc_mesh,
    compiler_params=_sc_params,
    scratch_types=[
        pltpu.VMEM((_QCH_R, 512), jnp.uint8),
        pltpu.VMEM((_QCH_R, 512), jnp.uint8),
        pltpu.VMEM((256 * _L,), jnp.int32),
        pltpu.VMEM((_L, 256), jnp.int32),
        pltpu.SemaphoreType.DMA,
        pltpu.SemaphoreType.DMA,
    ],
)
def _sc_hist_u8(q_hbm, out_hbm, buf0, buf1, hist2, histt, sem0, sem1):
    wid = lax.axis_index("s") * _NC + lax.axis_index("c")
    _zero_hist(hist2)

    bufs = (buf0, buf1)
    sems = (sem0, sem1)
    copies = [None, None]
    lane = lax.broadcasted_iota(jnp.int32, (_L,), 0)
    one = jnp.ones((_L,), jnp.int32)

    def _src(c):
        g = wid * _NQCH + c
        b = g // 12
        ch = (g % 12) // 4
        rb = g % 4
        return q_hbm.at[b, ch, pl.ds(rb * _QCH_R, _QCH_R), :]

    copies[0] = pltpu.async_copy(_src(0), buf0, sem0)
    for c in range(_NQCH):
        if c + 1 < _NQCH:
            nxt = (c + 1) % 2
            copies[nxt] = pltpu.async_copy(_src(c + 1), bufs[nxt], sems[nxt])
        copies[c % 2].wait()
        cur = bufs[c % 2]

        @plsc.parallel_loop(0, _QCHUNK, step=4 * _L, unroll=4)
        def _chunk_body(i):
            x64 = cur[jnp.right_shift(i, 9), pl.ds(i & 511, 4 * _L)]
            v = plsc.bitcast(x64, jnp.int32)
            s0 = (jnp.left_shift(v, 4) & 0xFF0) | lane
            s1 = (jnp.right_shift(v, 4) & 0xFF0) | lane
            s2 = (jnp.right_shift(v, 12) & 0xFF0) | lane
            s3 = (jnp.right_shift(v, 20) & 0xFF0) | lane
            plsc.addupdate_scatter(hist2, [s0], one)
            plsc.addupdate_scatter(hist2, [s1], one)
            plsc.addupdate_scatter(hist2, [s2], one)
            plsc.addupdate_scatter(hist2, [s3], one)

    _hist_epilogue(hist2, histt, out_hbm, wid, lane)


def _tc_msq_body(o_ref, i_ref, q_ref, sq_ref, acc):
    step = pl.program_id(0)

    @pl.when(step == 0)
    def _init():
        acc[0, 0] = 0.0

    o = o_ref[...]
    d = o - i_ref[...]
    acc[0, 0] += jnp.sum(d * d)
    q_ref[...] = (o * 256.0).astype(jnp.uint8)

    @pl.when(step == _HB - 1)
    def _fini():
        sq_ref[0, 0] = acc[0, 0]


def _tc_mse_body(o_ref, i_ref, sq_ref, acc):
    step = pl.program_id(0)

    @pl.when(step == 0)
    def _init():
        acc[0, 0] = 0.0

    d = o_ref[...] - i_ref[...]
    acc[0, 0] += jnp.sum(d * d)

    @pl.when(step == _HB - 1)
    def _fini():
        sq_ref[0, 0] = acc[0, 0]


def _tc_combine_body(h0_ref, h1_ref, sq0_ref, sq1_ref, loss_ref, bpp_ref, dist_ref):
    counts = jnp.sum(h0_ref[...].astype(jnp.float32), axis=0) + jnp.sum(
        h1_ref[...].astype(jnp.float32), axis=0
    )
    total = jnp.sum(counts)
    p = counts / total
    p = jnp.clip(p, 1e-12, 1.0)
    ent = -jnp.sum(p * jnp.log2(p))
    bpp = ent / 32.0
    dist = (sq0_ref[0, 0] + sq1_ref[0, 0]) / float(_N)
    bpp_ref[0, 0] = bpp
    dist_ref[0, 0] = dist
    loss_ref[0, 0] = bpp + dist


@jax.jit
def kernel(outputs, inputs):
    q1, sq1 = pl.pallas_call(
        _tc_msq_body,
        grid=(_HB,),
        in_specs=[
            pl.BlockSpec((1, 3, 512, 512), lambda g: (_HB + g, 0, 0, 0)),
            pl.BlockSpec((1, 3, 512, 512), lambda g: (_HB + g, 0, 0, 0)),
        ],
        out_specs=[
            pl.BlockSpec((1, 3, 512, 512), lambda g: (g, 0, 0, 0)),
            pl.BlockSpec(memory_space=pltpu.SMEM),
        ],
        out_shape=[
            jax.ShapeDtypeStruct((_HB, 3, 512, 512), jnp.uint8),
            jax.ShapeDtypeStruct((1, 1), jnp.float32),
        ],
        scratch_shapes=[pltpu.SMEM((1, 1), jnp.float32)],
    )(outputs, inputs)
    h0 = _sc_hist_f32(outputs)
    sq0 = pl.pallas_call(
        _tc_mse_body,
        grid=(_HB,),
        in_specs=[
            pl.BlockSpec((1, 3, 512, 512), lambda g: (g, 0, 0, 0)),
            pl.BlockSpec((1, 3, 512, 512), lambda g: (g, 0, 0, 0)),
        ],
        out_specs=pl.BlockSpec(memory_space=pltpu.SMEM),
        out_shape=jax.ShapeDtypeStruct((1, 1), jnp.float32),
        scratch_shapes=[pltpu.SMEM((1, 1), jnp.float32)],
    )(outputs, inputs)
    h1 = _sc_hist_u8(q1)
    loss, bpp, dist = pl.pallas_call(
        _tc_combine_body,
        in_specs=[
            pl.BlockSpec((_NW * _L, 256), lambda: (0, 0)),
            pl.BlockSpec((_NW * _L, 256), lambda: (0, 0)),
            pl.BlockSpec(memory_space=pltpu.SMEM),
            pl.BlockSpec(memory_space=pltpu.SMEM),
        ],
        out_specs=[
            pl.BlockSpec(memory_space=pltpu.SMEM),
            pl.BlockSpec(memory_space=pltpu.SMEM),
            pl.BlockSpec(memory_space=pltpu.SMEM),
        ],
        out_shape=[jax.ShapeDtypeStruct((1, 1), jnp.float32)] * 3,
    )(
        h0.reshape(_NW * _L, 256),
        h1.reshape(_NW * _L, 256),
        sq0,
        sq1,
    )
    return loss[0, 0], bpp[0, 0], dist[0, 0]


# speedup vs baseline: 1.0013x; 1.0013x over previous
import functools

import jax
import jax.numpy as jnp
from jax import lax
from jax.experimental import pallas as pl
from jax.experimental.pallas import tpu as pltpu
from jax.experimental.pallas import tpu_sc as plsc

_N = 32 * 3 * 512 * 512
_NC, _NS, _L = 2, 16, 16
_NW = _NC * _NS
_HB = 16

_FCH_R = 64
_FCHUNK = _FCH_R * 512
_NFCH = _HB * 3 * (512 // _FCH_R) // _NW

_QCH_R = 128
_QCHUNK = _QCH_R * 512
_NQCH = _HB * 3 * (512 // _QCH_R) // _NW


_sc_mesh = plsc.VectorSubcoreMesh(core_axis_name="c", subcore_axis_name="s")
_sc_params = pltpu.CompilerParams(
    needs_layout_passes=False, use_tc_tiling_on_sc=True
)


def _hist_epilogue(hist2, histt, out_hbm, wid, lane):
    @plsc.parallel_loop(0, _L * 256, step=_L, unroll=4)
    def _tr_body(j):
        lane_out = jnp.right_shift(j, 8)
        bin_base = j & 255
        src = jnp.left_shift(bin_base + lane, 4) | lane_out
        histt[lane_out, pl.ds(bin_base, _L)] = plsc.load_gather(hist2, [src])

    pltpu.sync_copy(histt, out_hbm.at[wid])


def _zero_hist(hist2):
    zero = jnp.zeros((_L,), jnp.int32)

    @plsc.parallel_loop(0, 256 * _L, step=_L)
    def _zero_body(r):
        hist2[pl.ds(r, _L)] = zero


@functools.partial(
    pl.kernel,
    out_type=jax.ShapeDtypeStruct((_NW, _L, 256), jnp.int32),
    mesh=_sc_mesh,
    compiler_params=_sc_params,
    scratch_types=[
        pltpu.VMEM((_FCH_R, 512), jnp.float32),
        pltpu.VMEM((_FCH_R, 512), jnp.float32),
        pltpu.VMEM((256 * _L,), jnp.int32),
        pltpu.VMEM((_L, 256), jnp.int32),
        pltpu.SemaphoreType.DMA,
        pltpu.SemaphoreType.DMA,
    ],
)
def _sc_hist_f32(x_hbm, out_hbm, buf0, buf1, hist2, histt, sem0, sem1):
    wid = lax.axis_index("s") * _NC + lax.axis_index("c")
    _zero_hist(hist2)

    bufs = (buf0, buf1)
    sems = (sem0, sem1)
    copies = [None, None]
    lane = lax.broadcasted_iota(jnp.int32, (_L,), 0)
    one = jnp.ones((_L,), jnp.int32)

    def _src(c):
        g = wid * _NFCH + c
        b = g // 24
        ch = (g % 24) // 8
        rb = g % 8
        return x_hbm.at[b, ch, pl.ds(rb * _FCH_R, _FCH_R), :]

    copies[0] = pltpu.async_copy(_src(0), buf0, sem0)
    for c in range(_NFCH):
        if c + 1 < _NFCH:
            nxt = (c + 1) % 2
            copies[nxt] = pltpu.async_copy(_src(c + 1), bufs[nxt], sems[nxt])
        copies[c % 2].wait()
        cur = bufs[c % 2]

        @plsc.parallel_loop(0, _FCHUNK, step=_L, unroll=8)
        def _chunk_body(i):
            x = cur[jnp.right_shift(i, 9), pl.ds(i & 511, _L)]
            idx = (x * 256.0).astype(jnp.int32)
            slot = jnp.left_shift(idx, 4) | lane
            plsc.addupdate_scatter(hist2, [slot], one)

    _hist_epilogue(hist2, histt, out_hbm, wid, lane)


@functools.partial(
    pl.kernel,
    out_type=jax.ShapeDtypeStruct((_NW, _L, 256), jnp.int32),
    mesh=_sc_mesh,
    compiler_params=_sc_params,
    scratch_types=[
        pltpu.VMEM((_QCH_R, 512), jnp.uint8),
        pltpu.VMEM((_QCH_R, 512), jnp.uint8),
        pltpu.VMEM((256 * _L,), jnp.int32),
        pltpu.VMEM((_L, 256), jnp.int32),
        pltpu.SemaphoreType.DMA,
        pltpu.SemaphoreType.DMA,
    ],
)
def _sc_hist_u8(q_hbm, out_hbm, buf0, buf1, hist2, histt, sem0, sem1):
    wid = lax.axis_index("s") * _NC + lax.axis_index("c")
    _zero_hist(hist2)

    bufs = (buf0, buf1)
    sems = (sem0, sem1)
    copies = [None, None]
    lane = lax.broadcasted_iota(jnp.int32, (_L,), 0)
    one = jnp.ones((_L,), jnp.int32)

    def _src(c):
        g = wid * _NQCH + c
        b = g // 12
        ch = (g % 12) // 4
        rb = g % 4
        return q_hbm.at[b, ch, pl.ds(rb * _QCH_R, _QCH_R), :]

    copies[0] = pltpu.async_copy(_src(0), buf0, sem0)
    for c in range(_NQCH):
        if c + 1 < _NQCH:
            nxt = (c + 1) % 2
            copies[nxt] = pltpu.async_copy(_src(c + 1), bufs[nxt], sems[nxt])
        copies[c % 2].wait()
        cur = bufs[c % 2]

        @plsc.parallel_loop(0, _QCHUNK, step=4 * _L, unroll=4)
        def _chunk_body(i):
            x64 = cur[jnp.right_shift(i, 9), pl.ds(i & 511, 4 * _L)]
            v = plsc.bitcast(x64, jnp.int32)
            s0 = (jnp.left_shift(v, 4) & 0xFF0) | lane
            s1 = (jnp.right_shift(v, 4) & 0xFF0) | lane
            s2 = (jnp.right_shift(v, 12) & 0xFF0) | lane
            s3 = (jnp.right_shift(v, 20) & 0xFF0) | lane
            plsc.addupdate_scatter(hist2, [s0], one)
            plsc.addupdate_scatter(hist2, [s1], one)
            plsc.addupdate_scatter(hist2, [s2], one)
            plsc.addupdate_scatter(hist2, [s3], one)

    _hist_epilogue(hist2, histt, out_hbm, wid, lane)


def _tc_msq_body(o_ref, i_ref, q_ref, sq_ref, acc):
    step = pl.program_id(0)

    @pl.when(step == 0)
    def _init():
        acc[0, 0] = 0.0

    o = o_ref[...]
    d = o - i_ref[...]
    acc[0, 0] += jnp.sum(d * d)
    q_ref[...] = (o * 256.0).astype(jnp.uint8)

    @pl.when(step == _HB - 1)
    def _fini():
        sq_ref[0, 0] = acc[0, 0]


def _tc_mse_body(o_ref, i_ref, sq_ref, acc):
    step = pl.program_id(0)

    @pl.when(step == 0)
    def _init():
        acc[0, 0] = 0.0

    d = o_ref[...] - i_ref[...]
    acc[0, 0] += jnp.sum(d * d)

    @pl.when(step == _HB - 1)
    def _fini():
        sq_ref[0, 0] = acc[0, 0]


def _tc_combine_body(h0_ref, h1_ref, sq0_ref, sq1_ref, loss_ref, bpp_ref, dist_ref):
    counts = jnp.sum(h0_ref[...].astype(jnp.float32), axis=0) + jnp.sum(
        h1_ref[...].astype(jnp.float32), axis=0
    )
    total = jnp.sum(counts)
    p = counts / total
    p = jnp.clip(p, 1e-12, 1.0)
    ent = -jnp.sum(p * jnp.log2(p))
    bpp = ent / 32.0
    dist = (sq0_ref[0, 0] + sq1_ref[0, 0]) / float(_N)
    bpp_ref[0, 0] = bpp
    dist_ref[0, 0] = dist
    loss_ref[0, 0] = bpp + dist


@jax.jit
def kernel(outputs, inputs):
    h0 = _sc_hist_f32(outputs)
    q1, sq1 = pl.pallas_call(
        _tc_msq_body,
        grid=(_HB,),
        in_specs=[
            pl.BlockSpec((1, 3, 512, 512), lambda g: (_HB + g, 0, 0, 0)),
            pl.BlockSpec((1, 3, 512, 512), lambda g: (_HB + g, 0, 0, 0)),
        ],
        out_specs=[
            pl.BlockSpec((1, 3, 512, 512), lambda g: (g, 0, 0, 0)),
            pl.BlockSpec(memory_space=pltpu.SMEM),
        ],
        out_shape=[
            jax.ShapeDtypeStruct((_HB, 3, 512, 512), jnp.uint8),
            jax.ShapeDtypeStruct((1, 1), jnp.float32),
        ],
        scratch_shapes=[pltpu.SMEM((1, 1), jnp.float32)],
    )(outputs, inputs)
    sq0 = pl.pallas_call(
        _tc_mse_body,
        grid=(_HB,),
        in_specs=[
            pl.BlockSpec((1, 3, 512, 512), lambda g: (g, 0, 0, 0)),
            pl.BlockSpec((1, 3, 512, 512), lambda g: (g, 0, 0, 0)),
        ],
        out_specs=pl.BlockSpec(memory_space=pltpu.SMEM),
        out_shape=jax.ShapeDtypeStruct((1, 1), jnp.float32),
        scratch_shapes=[pltpu.SMEM((1, 1), jnp.float32)],
    )(outputs, inputs)
    h1 = _sc_hist_u8(q1)
    loss, bpp, dist = pl.pallas_call(
        _tc_combine_body,
        in_specs=[
            pl.BlockSpec((_NW * _L, 256), lambda: (0, 0)),
            pl.BlockSpec((_NW * _L, 256), lambda: (0, 0)),
            pl.BlockSpec(memory_space=pltpu.SMEM),
            pl.BlockSpec(memory_space=pltpu.SMEM),
        ],
        out_specs=[
            pl.BlockSpec(memory_space=pltpu.SMEM),
            pl.BlockSpec(memory_space=pltpu.SMEM),
            pl.BlockSpec(memory_space=pltpu.SMEM),
        ],
        out_shape=[jax.ShapeDtypeStruct((1, 1), jnp.float32)] * 3,
    )(
        h0.reshape(_NW * _L, 256),
        h1.reshape(_NW * _L, 256),
        sq0,
        sq1,
    )
    return loss[0, 0], bpp[0, 0], dist[0, 0]


# speedup vs baseline: 1.0863x; 1.0848x over previous
import functools

import jax
import jax.numpy as jnp
from jax import lax
from jax.experimental import pallas as pl
from jax.experimental.pallas import tpu as pltpu
from jax.experimental.pallas import tpu_sc as plsc

_N = 32 * 3 * 512 * 512
_NC, _NS, _L = 2, 16, 16
_NW = _NC * _NS
_CH_R = 32
_CHUNK = _CH_R * 512
_NCHUNK = 3 * 512 // _CH_R
_NMSE = _NCHUNK // 3


_sc_mesh = plsc.VectorSubcoreMesh(core_axis_name="c", subcore_axis_name="s")


@functools.partial(
    pl.kernel,
    out_type=[
        jax.ShapeDtypeStruct((_NW, _L, 256), jnp.int32),
        jax.ShapeDtypeStruct((_NW, _L), jnp.float32),
    ],
    mesh=_sc_mesh,
    compiler_params=pltpu.CompilerParams(
        needs_layout_passes=False, use_tc_tiling_on_sc=True
    ),
    scratch_types=[
        pltpu.VMEM((_CH_R, 512), jnp.float32),
        pltpu.VMEM((_CH_R, 512), jnp.float32),
        pltpu.VMEM((_CH_R, 512), jnp.float32),
        pltpu.VMEM((_CH_R, 512), jnp.float32),
        pltpu.VMEM((256 * _L,), jnp.int32),
        pltpu.VMEM((_L, 256), jnp.int32),
        pltpu.VMEM((_L,), jnp.float32),
        pltpu.SemaphoreType.DMA,
        pltpu.SemaphoreType.DMA,
        pltpu.SemaphoreType.DMA,
        pltpu.SemaphoreType.DMA,
    ],
)
def _sc_hist(
    x_hbm,
    in_hbm,
    out_hbm,
    msq_hbm,
    obuf0,
    obuf1,
    ibuf0,
    ibuf1,
    hist2,
    histt,
    maccv,
    sem0,
    sem1,
    sem2,
    sem3,
):
    wid = lax.axis_index("s") * _NC + lax.axis_index("c")

    zero = jnp.zeros((_L,), jnp.int32)

    @plsc.parallel_loop(0, 256 * _L, step=_L)
    def _zero_body(r):
        hist2[pl.ds(r, _L)] = zero

    obufs = (obuf0, obuf1)
    ibufs = (ibuf0, ibuf1)
    osems = (sem0, sem1)
    isems = (sem2, sem3)
    ocopies = [None, None]
    icopies = [None, None]
    lane = lax.broadcasted_iota(jnp.int32, (_L,), 0)
    one = jnp.ones((_L,), jnp.int32)

    def _src(arr, c):
        ch = c // _NMSE
        rb = c % _NMSE
        return arr.at[wid, ch, pl.ds(rb * _CH_R, _CH_R), :]

    ocopies[0] = pltpu.async_copy(_src(x_hbm, 0), obuf0, sem0)
    icopies[0] = pltpu.async_copy(_src(in_hbm, 0), ibuf0, sem2)
    acc = jnp.zeros((_L,), jnp.float32)
    for c in range(_NCHUNK):
        if c + 1 < _NCHUNK:
            nxt = (c + 1) % 2
            ocopies[nxt] = pltpu.async_copy(
                _src(x_hbm, c + 1), obufs[nxt], osems[nxt]
            )
            if c + 1 < _NMSE:
                icopies[nxt] = pltpu.async_copy(
                    _src(in_hbm, c + 1), ibufs[nxt], isems[nxt]
                )
        ocopies[c % 2].wait()
        cur = obufs[c % 2]

        if c < _NMSE:
            icopies[c % 2].wait()
            icur = ibufs[c % 2]

            @plsc.parallel_loop(0, _CHUNK, step=_L, unroll=8, carry=acc)
            def _mse_body(i, a):
                row = jnp.right_shift(i, 9)
                col = i & 511
                x = cur[row, pl.ds(col, _L)]
                idx = (x * 256.0).astype(jnp.int32)
                slot = jnp.left_shift(idx, 4) | lane
                plsc.addupdate_scatter(hist2, [slot], one)
                d = x - icur[row, pl.ds(col, _L)]
                return a + d * d

            acc = _mse_body
        else:

            @plsc.parallel_loop(0, _CHUNK, step=_L, unroll=8)
            def _chunk_body(i):
                x = cur[jnp.right_shift(i, 9), pl.ds(i & 511, _L)]
                idx = (x * 256.0).astype(jnp.int32)
                slot = jnp.left_shift(idx, 4) | lane
                plsc.addupdate_scatter(hist2, [slot], one)

    maccv[...] = acc
    pltpu.sync_copy(maccv, msq_hbm.at[wid])

    @plsc.parallel_loop(0, _L * 256, step=_L, unroll=4)
    def _tr_body(j):
        lane_out = jnp.right_shift(j, 8)
        bin_base = j & 255
        src = jnp.left_shift(bin_base + lane, 4) | lane_out
        histt[lane_out, pl.ds(bin_base, _L)] = plsc.load_gather(hist2, [src])

    pltpu.sync_copy(histt, out_hbm.at[wid])


def _tc_mse_body(o_ref, i_ref, sq_ref, acc):
    b = pl.program_id(0)
    j = pl.program_id(1)

    @pl.when((b == 0) & (j == 0))
    def _init():
        acc[0, 0] = 0.0

    d = o_ref[...] - i_ref[...]
    acc[0, 0] += jnp.sum(d * d)

    @pl.when((b == 31) & (j == 1))
    def _fini():
        sq_ref[0, 0] = acc[0, 0]


def _tc_combine_body(hist_ref, msq_ref, sq_ref, loss_ref, bpp_ref, dist_ref):
    counts = jnp.sum(hist_ref[...].astype(jnp.float32), axis=0)
    total = jnp.sum(counts)
    p = counts / total
    p = jnp.clip(p, 1e-12, 1.0)
    ent = -jnp.sum(p * jnp.log2(p))
    bpp = ent / 32.0
    dist = (sq_ref[0, 0] + jnp.sum(msq_ref[...])) / float(_N)
    bpp_ref[0, 0] = bpp
    dist_ref[0, 0] = dist
    loss_ref[0, 0] = bpp + dist


@jax.jit
def kernel(outputs, inputs):
    hist, msq = _sc_hist(outputs, inputs)
    sq = pl.pallas_call(
        _tc_mse_body,
        grid=(32, 2),
        in_specs=[
            pl.BlockSpec((1, 1, 512, 512), lambda b, j: (b, j + 1, 0, 0)),
            pl.BlockSpec((1, 1, 512, 512), lambda b, j: (b, j + 1, 0, 0)),
        ],
        out_specs=pl.BlockSpec(memory_space=pltpu.SMEM),
        out_shape=jax.ShapeDtypeStruct((1, 1), jnp.float32),
        scratch_shapes=[pltpu.SMEM((1, 1), jnp.float32)],
    )(outputs, inputs)
    loss, bpp, dist = pl.pallas_call(
        _tc_combine_body,
        in_specs=[
            pl.BlockSpec((_NW * _L, 256), lambda: (0, 0)),
            pl.BlockSpec((_NW, _L), lambda: (0, 0)),
            pl.BlockSpec(memory_space=pltpu.SMEM),
        ],
        out_specs=[
            pl.BlockSpec(memory_space=pltpu.SMEM),
            pl.BlockSpec(memory_space=pltpu.SMEM),
            pl.BlockSpec(memory_space=pltpu.SMEM),
        ],
        out_shape=[jax.ShapeDtypeStruct((1, 1), jnp.float32)] * 3,
    )(hist.reshape(_NW * _L, 256), msq, sq)
    return loss[0, 0], bpp[0, 0], dist[0, 0]


# speedup vs baseline: 1.1474x; 1.0562x over previous
import functools

import jax
import jax.numpy as jnp
from jax import lax
from jax.experimental import pallas as pl
from jax.experimental.pallas import tpu as pltpu
from jax.experimental.pallas import tpu_sc as plsc

_N = 32 * 3 * 512 * 512
_NC, _NS, _L = 2, 16, 16
_NW = _NC * _NS
_PER_W = _N // _NW
_CHUNK = 32768
_NCHUNK = _PER_W // _CHUNK
_VECS = _CHUNK // _L

_ROWS = _N // 1024
_BR = 2048
_G = _ROWS // _BR


_sc_mesh = plsc.VectorSubcoreMesh(core_axis_name="c", subcore_axis_name="s")


@functools.partial(
    pl.kernel,
    out_type=jax.ShapeDtypeStruct((_NW, _L, 256), jnp.int32),
    mesh=_sc_mesh,
    compiler_params=pltpu.CompilerParams(
        needs_layout_passes=False, use_tc_tiling_on_sc=True
    ),
    scratch_types=[
        pltpu.VMEM((64, 512), jnp.float32),
        pltpu.VMEM((64, 512), jnp.float32),
        pltpu.VMEM((256 * _L,), jnp.int32),
        pltpu.VMEM((_L, 256), jnp.int32),
        pltpu.SemaphoreType.DMA,
        pltpu.SemaphoreType.DMA,
    ],
)
def _sc_hist(x_hbm, out_hbm, buf0, buf1, hist2, histt, sem0, sem1):
    wid = lax.axis_index("s") * _NC + lax.axis_index("c")

    zero = jnp.zeros((_L,), jnp.int32)

    @plsc.parallel_loop(0, 256 * _L, step=_L)
    def _zero_body(r):
        hist2[pl.ds(r, _L)] = zero

    bufs = (buf0, buf1)
    sems = (sem0, sem1)
    copies = [None, None]
    lane = lax.broadcasted_iota(jnp.int32, (_L,), 0)
    one = jnp.ones((_L,), jnp.int32)

    copies[0] = pltpu.async_copy(
        x_hbm.at[wid, 0, pl.ds(0, 64), :], buf0, sem0
    )
    for c in range(_NCHUNK):
        if c + 1 < _NCHUNK:
            nxt = (c + 1) % 2
            ch, r0 = divmod(c + 1, 8)
            copies[nxt] = pltpu.async_copy(
                x_hbm.at[wid, ch, pl.ds(r0 * 64, 64), :],
                bufs[nxt],
                sems[nxt],
            )
        copies[c % 2].wait()
        cur = bufs[c % 2]

        @plsc.parallel_loop(0, _CHUNK, step=_L, unroll=8)
        def _chunk_body(i):
            x = cur[jnp.right_shift(i, 9), pl.ds(i & 511, _L)]
            idx = (x * 256.0).astype(jnp.int32)
            slot = jnp.left_shift(idx, 4) | lane
            plsc.addupdate_scatter(hist2, [slot], one)

    @plsc.parallel_loop(0, _L * 256, step=_L, unroll=4)
    def _tr_body(j):
        lane_out = jnp.right_shift(j, 8)
        bin_base = j & 255
        src = jnp.left_shift(bin_base + lane, 4) | lane_out
        histt[lane_out, pl.ds(bin_base, _L)] = plsc.load_gather(hist2, [src])

    pltpu.sync_copy(histt, out_hbm.at[wid])


def _tc_mse_body(o_ref, i_ref, sq_ref, acc):
    step = pl.program_id(0)

    @pl.when(step == 0)
    def _init():
        acc[0, 0] = 0.0

    d = o_ref[...] - i_ref[...]
    acc[0, 0] += jnp.sum(d * d)

    @pl.when(step == 31)
    def _fini():
        sq_ref[0, 0] = acc[0, 0]


def _tc_combine_body(hist_ref, sq_ref, loss_ref, bpp_ref, dist_ref):
    counts = jnp.sum(hist_ref[...].astype(jnp.float32), axis=0)
    total = jnp.sum(counts)
    p = counts / total
    p = jnp.clip(p, 1e-12, 1.0)
    ent = -jnp.sum(p * jnp.log2(p))
    bpp = ent / 32.0
    dist = sq_ref[0, 0] / float(_N)
    bpp_ref[0, 0] = bpp
    dist_ref[0, 0] = dist
    loss_ref[0, 0] = bpp + dist


@jax.jit
def kernel(outputs, inputs):
    hist = _sc_hist(outputs)
    sq = pl.pallas_call(
        _tc_mse_body,
        grid=(32,),
        in_specs=[
            pl.BlockSpec((1, 3, 512, 512), lambda i: (i, 0, 0, 0)),
            pl.BlockSpec((1, 3, 512, 512), lambda i: (i, 0, 0, 0)),
        ],
        out_specs=pl.BlockSpec(memory_space=pltpu.SMEM),
        out_shape=jax.ShapeDtypeStruct((1, 1), jnp.float32),
        scratch_shapes=[pltpu.SMEM((1, 1), jnp.float32)],
    )(outputs, inputs)
    loss, bpp, dist = pl.pallas_call(
        _tc_combine_body,
        in_specs=[
            pl.BlockSpec((_NW * _L, 256), lambda: (0, 0)),
            pl.BlockSpec(memory_space=pltpu.SMEM),
        ],
        out_specs=[
            pl.BlockSpec(memory_space=pltpu.SMEM),
            pl.BlockSpec(memory_space=pltpu.SMEM),
            pl.BlockSpec(memory_space=pltpu.SMEM),
        ],
        out_shape=[jax.ShapeDtypeStruct((1, 1), jnp.float32)] * 3,
    )(hist.reshape(_NW * _L, 256), sq)
    return loss[0, 0], bpp[0, 0], dist[0, 0]
